# baseline (speedup 1.0000x reference)
import jax
import jax.numpy as jnp
from jax import lax
from jax.experimental import pallas as pl
from jax.experimental.pallas import tpu as pltpu
from jax.experimental.pallas import tpu_sc as plsc

BIN_NUM = 5
NUM_GENES = 16906
EMBED_DIM = 200
BATCH = 8
EPS = 1e-6

LBLK = 2048
GRID = (NUM_GENES + LBLK - 1) // LBLK
LPAD = GRID * LBLK
TAIL_PID = GRID - 1

WBIG = 640
WSML = 512
NBIG = 4
COVER = NBIG * WBIG + (32 - NBIG) * WSML


def _table_blk(x_ref, tw_ref, g2v_ref, gdp_ref, id_ref, st_ref):
    x = x_ref[...]
    x = jnp.where(jnp.isnan(x), 0.0, x)
    x = jnp.clip(x, 0.0, float(BIN_NUM))
    ids_f = lax.round(x, lax.RoundingMethod.TO_NEAREST_EVEN)
    ids_i = ids_f.astype(jnp.int32)
    id_ref[...] = ids_i
    tw = tw_ref[...]
    g2v = g2v_ref[...]
    ntw = jnp.sum(tw * tw, axis=1, keepdims=True)
    ng = jnp.sum(g2v * g2v, axis=1)[None, :]
    gd = lax.dot_general(tw, g2v, (((1,), (1,)), ((), ())),
                         preferred_element_type=jnp.float32)
    gdp_ref[...] = 2.0 * gd + ntw + ng

    @pl.when(pl.program_id(0) == TAIL_PID)
    def _tail_scales():
        kiota = lax.broadcasted_iota(jnp.int32, (BATCH, LBLK, 8), 2)
        onehot = (ids_i[:, :, None] == kiota).astype(jnp.float32)
        for b in range(BATCH):
            te = jnp.dot(onehot[b], tw, preferred_element_type=jnp.float32)
            h = te + g2v
            ms = jnp.mean(h * h, axis=-1)
            st_ref[b, :] = lax.rsqrt(ms + EPS)


def _table(x, tw8, g2v):
    return pl.pallas_call(
        _table_blk,
        grid=(GRID,),
        in_specs=[
            pl.BlockSpec((BATCH, LBLK), lambda i: (0, i)),
            pl.BlockSpec((8, EMBED_DIM), lambda i: (0, 0)),
            pl.BlockSpec((LBLK, EMBED_DIM), lambda i: (i, 0)),
        ],
        out_specs=[
            pl.BlockSpec((8, LBLK), lambda i: (0, i)),
            pl.BlockSpec((BATCH, LBLK), lambda i: (0, i)),
            pl.BlockSpec((BATCH, LBLK), lambda i: (0, 0)),
        ],
        out_shape=[
            jax.ShapeDtypeStruct((8, LPAD), jnp.float32),
            jax.ShapeDtypeStruct((BATCH, LPAD), jnp.int32),
            jax.ShapeDtypeStruct((BATCH, LBLK), jnp.float32),
        ],
    )(x, tw8, g2v)


def _rsqrt_nr(y):
    i = plsc.bitcast(y, jnp.int32)
    i = 0x5F3759DF - lax.shift_right_logical(i, 1)
    r = plsc.bitcast(i, jnp.float32)
    for _ in range(3):
        r = r * (1.5 - 0.5 * y * r * r)
    return r


def _sc_scales_body(gdp_hbm, id_hbm, s_hbm, gd_v, id_v, s_v):
    wid = lax.axis_index("s") * 2 + lax.axis_index("c")

    def run(c0, w):
        for k in range(6):
            pltpu.sync_copy(gdp_hbm.at[k, pl.ds(c0, w)],
                            gd_v.at[pl.ds(k * WBIG, w)])
        for b in range(BATCH):
            pltpu.sync_copy(id_hbm.at[b, pl.ds(c0, w)],
                            id_v.at[pl.ds(b * WBIG, w)])

        def b_body(b, carry):
            @plsc.parallel_loop(0, w // 16, unroll=4)
            def j_body(j):
                col = j * 16
                idv = id_v[pl.ds(b * WBIG + col, 16)]
                acc = jnp.zeros((16,), jnp.float32)
                for k in range(6):
                    gk = gd_v[pl.ds(k * WBIG + col, 16)]
                    acc = jnp.where(idv == k, gk, acc)
                y = acc * (1.0 / EMBED_DIM) + EPS
                s_v[pl.ds(b * WBIG + col, 16)] = _rsqrt_nr(y)

            return carry

        lax.fori_loop(0, BATCH, b_body, 0)
        for b in range(BATCH):
            pltpu.sync_copy(s_v.at[pl.ds(b * WBIG, w)],
                            s_hbm.at[b, pl.ds(c0, w)])

    @pl.when(wid < NBIG)
    def _big():
        run(pl.multiple_of(wid * WBIG, 128), WBIG)

    @pl.when(wid >= NBIG)
    def _small():
        run(pl.multiple_of(NBIG * WBIG + (wid - NBIG) * WSML, 128), WSML)


def _sc_scales(gdp, ids):
    mesh = plsc.VectorSubcoreMesh(core_axis_name="c", subcore_axis_name="s")
    return pl.kernel(
        _sc_scales_body,
        out_type=jax.ShapeDtypeStruct((BATCH, LPAD), jnp.float32),
        mesh=mesh,
        compiler_params=pltpu.CompilerParams(use_tc_tiling_on_sc=False,
                                             needs_layout_passes=False),
        scratch_types=[
            pltpu.VMEM((6 * WBIG,), jnp.float32),
            pltpu.VMEM((BATCH * WBIG,), jnp.int32),
            pltpu.VMEM((BATCH * WBIG,), jnp.float32),
        ],
    )(gdp, ids)


def _main_blk(id_ref, tw_ref, g2v_ref, w_ref, s_ref, st_ref, out_ref):
    ids_i = id_ref[...]
    kiota = lax.broadcasted_iota(jnp.int32, (BATCH, LBLK, 8), 2)
    onehot = (ids_i[:, :, None] == kiota).astype(jnp.float32)
    tw = tw_ref[...]
    g2v = g2v_ref[...]
    w = w_ref[...]
    is_tail = pl.program_id(0) == TAIL_PID
    s_blk = jnp.where(is_tail, st_ref[...], s_ref[...])
    for b in range(BATCH):
        te = jnp.dot(onehot[b], tw, preferred_element_type=jnp.float32)
        h = te + g2v
        out_ref[b, :, :] = h * s_blk[b][:, None] * w


def _main(ids, tw8, g2v, w2d, s, st):
    return pl.pallas_call(
        _main_blk,
        grid=(GRID,),
        in_specs=[
            pl.BlockSpec((BATCH, LBLK), lambda i: (0, i)),
            pl.BlockSpec((8, EMBED_DIM), lambda i: (0, 0)),
            pl.BlockSpec((LBLK, EMBED_DIM), lambda i: (i, 0)),
            pl.BlockSpec((1, EMBED_DIM), lambda i: (0, 0)),
            pl.BlockSpec((BATCH, LBLK), lambda i: (0, i)),
            pl.BlockSpec((BATCH, LBLK), lambda i: (0, 0)),
        ],
        out_specs=pl.BlockSpec((BATCH, LBLK, EMBED_DIM), lambda i: (0, i, 0)),
        out_shape=jax.ShapeDtypeStruct((BATCH, NUM_GENES, EMBED_DIM),
                                       jnp.float32),
    )(ids, tw8, g2v, w2d, s, st)


def kernel(x, token_weight, gene2vec_weight, rms_weight):
    tw8 = jnp.concatenate(
        [token_weight, jnp.zeros((1, EMBED_DIM), token_weight.dtype)], axis=0
    )
    w2d = rms_weight.reshape(1, EMBED_DIM)
    gdp, ids, st = _table(x, tw8, gene2vec_weight)
    s = _sc_scales(gdp, ids)
    return _main(ids, tw8, gene2vec_weight, w2d, s, st)


# speedup vs baseline: 1.0015x; 1.0015x over previous
import jax
import jax.numpy as jnp
from jax import lax
from jax.experimental import pallas as pl
from jax.experimental.pallas import tpu as pltpu
from jax.experimental.pallas import tpu_sc as plsc

BIN_NUM = 5
NUM_GENES = 16906
EMBED_DIM = 200
BATCH = 8
EPS = 1e-6

LBLK = 1024
GRID = (NUM_GENES + LBLK - 1) // LBLK
LPAD = GRID * LBLK
TAIL_PID = GRID - 1

WBIG = 640
WSML = 512
NBIG = 4
COVER = NBIG * WBIG + (32 - NBIG) * WSML


def _table_blk(x_ref, tw_ref, g2v_ref, gdp_ref, id_ref, st_ref):
    x = x_ref[...]
    x = jnp.where(jnp.isnan(x), 0.0, x)
    x = jnp.clip(x, 0.0, float(BIN_NUM))
    ids_f = lax.round(x, lax.RoundingMethod.TO_NEAREST_EVEN)
    ids_i = ids_f.astype(jnp.int32)
    id_ref[...] = ids_i
    tw = tw_ref[...]
    g2v = g2v_ref[...]
    ntw = jnp.sum(tw * tw, axis=1, keepdims=True)
    ng = jnp.sum(g2v * g2v, axis=1)[None, :]
    gd = lax.dot_general(tw, g2v, (((1,), (1,)), ((), ())),
                         preferred_element_type=jnp.float32)
    gdp_ref[...] = 2.0 * gd + ntw + ng

    @pl.when(pl.program_id(0) == TAIL_PID)
    def _tail_scales():
        kiota = lax.broadcasted_iota(jnp.int32, (BATCH, LBLK, 8), 2)
        onehot = (ids_i[:, :, None] == kiota).astype(jnp.float32)
        for b in range(BATCH):
            te = jnp.dot(onehot[b], tw, preferred_element_type=jnp.float32)
            h = te + g2v
            ms = jnp.mean(h * h, axis=-1)
            st_ref[b, :] = lax.rsqrt(ms + EPS)


def _table(x, tw8, g2v):
    return pl.pallas_call(
        _table_blk,
        grid=(GRID,),
        in_specs=[
            pl.BlockSpec((BATCH, LBLK), lambda i: (0, i)),
            pl.BlockSpec((8, EMBED_DIM), lambda i: (0, 0)),
            pl.BlockSpec((LBLK, EMBED_DIM), lambda i: (i, 0)),
        ],
        out_specs=[
            pl.BlockSpec((8, LBLK), lambda i: (0, i)),
            pl.BlockSpec((BATCH, LBLK), lambda i: (0, i)),
            pl.BlockSpec((BATCH, LBLK), lambda i: (0, 0)),
        ],
        out_shape=[
            jax.ShapeDtypeStruct((8, LPAD), jnp.float32),
            jax.ShapeDtypeStruct((BATCH, LPAD), jnp.int32),
            jax.ShapeDtypeStruct((BATCH, LBLK), jnp.float32),
        ],
    )(x, tw8, g2v)


def _rsqrt_nr(y):
    i = plsc.bitcast(y, jnp.int32)
    i = 0x5F3759DF - lax.shift_right_logical(i, 1)
    r = plsc.bitcast(i, jnp.float32)
    for _ in range(3):
        r = r * (1.5 - 0.5 * y * r * r)
    return r


def _sc_scales_body(gdp_hbm, id_hbm, s_hbm, gd_v, id_v, s_v):
    wid = lax.axis_index("s") * 2 + lax.axis_index("c")

    def run(c0, w):
        for k in range(6):
            pltpu.sync_copy(gdp_hbm.at[k, pl.ds(c0, w)],
                            gd_v.at[pl.ds(k * WBIG, w)])
        for b in range(BATCH):
            pltpu.sync_copy(id_hbm.at[b, pl.ds(c0, w)],
                            id_v.at[pl.ds(b * WBIG, w)])

        def b_body(b, carry):
            @plsc.parallel_loop(0, w // 16, unroll=4)
            def j_body(j):
                col = j * 16
                idv = id_v[pl.ds(b * WBIG + col, 16)]
                acc = jnp.zeros((16,), jnp.float32)
                for k in range(6):
                    gk = gd_v[pl.ds(k * WBIG + col, 16)]
                    acc = jnp.where(idv == k, gk, acc)
                y = jnp.maximum(acc * (1.0 / EMBED_DIM), 0.0) + EPS
                s_v[pl.ds(b * WBIG + col, 16)] = _rsqrt_nr(y)

            return carry

        lax.fori_loop(0, BATCH, b_body, 0)
        for b in range(BATCH):
            pltpu.sync_copy(s_v.at[pl.ds(b * WBIG, w)],
                            s_hbm.at[b, pl.ds(c0, w)])

    @pl.when(wid < NBIG)
    def _big():
        run(pl.multiple_of(wid * WBIG, 128), WBIG)

    @pl.when(wid >= NBIG)
    def _small():
        run(pl.multiple_of(NBIG * WBIG + (wid - NBIG) * WSML, 128), WSML)


def _sc_scales(gdp, ids):
    mesh = plsc.VectorSubcoreMesh(core_axis_name="c", subcore_axis_name="s")
    return pl.kernel(
        _sc_scales_body,
        out_type=jax.ShapeDtypeStruct((BATCH, LPAD), jnp.float32),
        mesh=mesh,
        compiler_params=pltpu.CompilerParams(use_tc_tiling_on_sc=False,
                                             needs_layout_passes=False),
        scratch_types=[
            pltpu.VMEM((6 * WBIG,), jnp.float32),
            pltpu.VMEM((BATCH * WBIG,), jnp.int32),
            pltpu.VMEM((BATCH * WBIG,), jnp.float32),
        ],
    )(gdp, ids)


def _main_blk(id_ref, tw_ref, g2v_ref, w_ref, s_ref, st_ref, out_ref):
    ids_i = id_ref[...]
    kiota = lax.broadcasted_iota(jnp.int32, (BATCH, LBLK, 8), 2)
    onehot = (ids_i[:, :, None] == kiota).astype(jnp.float32)
    tw = tw_ref[...]
    g2v = g2v_ref[...]
    w = w_ref[...]
    is_tail = pl.program_id(0) == TAIL_PID
    s_blk = jnp.where(is_tail, st_ref[...], s_ref[...])
    for b in range(BATCH):
        te = jnp.dot(onehot[b], tw, preferred_element_type=jnp.float32)
        h = te + g2v
        out_ref[b, :, :] = h * s_blk[b][:, None] * w


def _main(ids, tw8, g2v, w2d, s, st):
    return pl.pallas_call(
        _main_blk,
        grid=(GRID,),
        in_specs=[
            pl.BlockSpec((BATCH, LBLK), lambda i: (0, i)),
            pl.BlockSpec((8, EMBED_DIM), lambda i: (0, 0)),
            pl.BlockSpec((LBLK, EMBED_DIM), lambda i: (i, 0)),
            pl.BlockSpec((1, EMBED_DIM), lambda i: (0, 0)),
            pl.BlockSpec((BATCH, LBLK), lambda i: (0, i)),
            pl.BlockSpec((BATCH, LBLK), lambda i: (0, 0)),
        ],
        out_specs=pl.BlockSpec((BATCH, LBLK, EMBED_DIM), lambda i: (0, i, 0)),
        out_shape=jax.ShapeDtypeStruct((BATCH, NUM_GENES, EMBED_DIM),
                                       jnp.float32),
    )(ids, tw8, g2v, w2d, s, st)


def kernel(x, token_weight, gene2vec_weight, rms_weight):
    tw8 = jnp.concatenate(
        [token_weight, jnp.zeros((1, EMBED_DIM), token_weight.dtype)], axis=0
    )
    w2d = rms_weight.reshape(1, EMBED_DIM)
    gdp, ids, st = _table(x, tw8, gene2vec_weight)
    s = _sc_scales(gdp, ids)
    return _main(ids, tw8, gene2vec_weight, w2d, s, st)
